# baseline (speedup 1.0000x reference)
import jax
import jax.numpy as jnp
from jax.experimental import pallas as pl

B = 16384
Z_DIM = 256
H_DIM = 512
N_VARS = 32
GAMMA_DIM = 64
EPS = 1e-6
BM = 1024

_HALF_LOG_2PI = 0.5 * float(jnp.log(2.0 * jnp.pi))


def _body(z_ref, x_ref, miss_ref, wh_ref, bh_ref, wg_ref, bg_ref,
          whead_ref, bhead_ref, nm_ref, nv_ref,
          lp_ref, lpm_ref, mean_ref, var_ref):
    z = z_ref[...]
    h = jnp.maximum(
        jnp.dot(z, wh_ref[...], preferred_element_type=jnp.float32)
        + bh_ref[...], 0.0)
    gamma = jnp.dot(h, wg_ref[...], preferred_element_type=jnp.float32) \
        + bg_ref[...]
    raw = jnp.dot(gamma, whead_ref[...], preferred_element_type=jnp.float32) \
        + bhead_ref[...]
    mean_raw = raw[:, :N_VARS]
    var_raw = raw[:, N_VARS:]
    sp = jnp.maximum(var_raw, 0.0) + jnp.log1p(jnp.exp(-jnp.abs(var_raw)))
    est_var0 = jnp.clip(sp, EPS, 1e20)
    data_mean = nm_ref[...]
    data_var = jnp.clip(nv_ref[...], EPS, 1e20)
    est_mean = jnp.sqrt(data_var) * mean_raw + data_mean
    est_var = data_var * est_var0
    diff = x_ref[...] - est_mean
    log_normal = (-0.5 * diff * diff / est_var
                  - _HALF_LOG_2PI - 0.5 * jnp.log(est_var))
    maskf = (miss_ref[...] == 1).astype(jnp.float32)
    lp_ref[...] = log_normal * maskf
    lpm_ref[...] = log_normal * (1.0 - maskf)
    mean_ref[...] = est_mean
    var_ref[...] = est_var


def kernel(z, batch_x, miss_list, norm_params, Wh, bh, Wg, bg, Wm, bm, Wv, bv):
    eye = jnp.eye(N_VARS, dtype=jnp.float32)
    wm_bd = (Wm[:, :, None] * eye[:, None, :]).reshape(N_VARS * GAMMA_DIM,
                                                       N_VARS)
    wv_bd = (Wv[:, :, None] * eye[:, None, :]).reshape(N_VARS * GAMMA_DIM,
                                                       N_VARS)
    whead = jnp.concatenate([wm_bd, wv_bd], axis=1)
    bhead = jnp.concatenate([bm, bv]).reshape(1, 2 * N_VARS)

    grid = (B // BM,)
    row = lambda i: (i, 0)
    const = lambda i: (0, 0)
    out_specs = [pl.BlockSpec((BM, N_VARS), row) for _ in range(4)]
    out_shapes = [jax.ShapeDtypeStruct((B, N_VARS), jnp.float32)
                  for _ in range(4)]

    lp, lpm, est_mean, est_var = pl.pallas_call(
        _body,
        grid=grid,
        in_specs=[
            pl.BlockSpec((BM, Z_DIM), row),
            pl.BlockSpec((BM, N_VARS), row),
            pl.BlockSpec((BM, N_VARS), row),
            pl.BlockSpec((Z_DIM, H_DIM), const),
            pl.BlockSpec((1, H_DIM), const),
            pl.BlockSpec((H_DIM, N_VARS * GAMMA_DIM), const),
            pl.BlockSpec((1, N_VARS * GAMMA_DIM), const),
            pl.BlockSpec((N_VARS * GAMMA_DIM, 2 * N_VARS), const),
            pl.BlockSpec((1, 2 * N_VARS), const),
            pl.BlockSpec((1, N_VARS), const),
            pl.BlockSpec((1, N_VARS), const),
        ],
        out_specs=out_specs,
        out_shape=out_shapes,
    )(z, batch_x, miss_list,
      Wh, bh.reshape(1, H_DIM), Wg, bg.reshape(1, N_VARS * GAMMA_DIM),
      whead, bhead,
      norm_params[:, 0].reshape(1, N_VARS), norm_params[:, 1].reshape(1, N_VARS))

    samples_x = est_mean
    params_x = jnp.stack([est_mean, est_var], axis=-1)
    return (lp, lpm, samples_x, params_x)


# speedup vs baseline: 1.4715x; 1.4715x over previous
import math

import jax
import jax.numpy as jnp
from jax.experimental import pallas as pl

B = 16384
Z_DIM = 256
H_DIM = 512
N_VARS = 32
GAMMA_DIM = 64
EPS = 1e-6
BM = 1024

_HALF_LOG_2PI = 0.5 * math.log(2.0 * math.pi)


def _fold_body(wg_ref, whead_ref, bg_ref, bias_ref, w2_ref, bhead_ref):
    w2_ref[...] = jnp.dot(wg_ref[...], whead_ref[...],
                          preferred_element_type=jnp.float32)
    bhead_ref[...] = jnp.dot(bg_ref[...], whead_ref[...],
                             preferred_element_type=jnp.float32) + bias_ref[...]


def _body(z_ref, x_ref, miss_ref, wh_ref, bh_ref, w2_ref, bhead_ref,
          nm_ref, nv_ref, lp_ref, lpm_ref, mean_ref, var_ref):
    z = z_ref[...]
    h = jnp.maximum(
        jnp.dot(z, wh_ref[...], preferred_element_type=jnp.float32)
        + bh_ref[...], 0.0)
    raw = jnp.dot(h, w2_ref[...], preferred_element_type=jnp.float32) \
        + bhead_ref[...]
    mean_raw = raw[:, :N_VARS]
    var_raw = raw[:, N_VARS:]
    sp = jnp.maximum(var_raw, 0.0) + jnp.log1p(jnp.exp(-jnp.abs(var_raw)))
    est_var0 = jnp.clip(sp, EPS, 1e20)
    data_mean = nm_ref[...]
    data_var = jnp.clip(nv_ref[...], EPS, 1e20)
    est_mean = jnp.sqrt(data_var) * mean_raw + data_mean
    est_var = data_var * est_var0
    diff = x_ref[...] - est_mean
    log_normal = (-0.5 * diff * diff / est_var
                  - _HALF_LOG_2PI - 0.5 * jnp.log(est_var))
    maskf = (miss_ref[...] == 1).astype(jnp.float32)
    lp_ref[...] = log_normal * maskf
    lpm_ref[...] = log_normal * (1.0 - maskf)
    mean_ref[...] = est_mean
    var_ref[...] = est_var


def kernel(z, batch_x, miss_list, norm_params, Wh, bh, Wg, bg, Wm, bm, Wv, bv):
    eye = jnp.eye(N_VARS, dtype=jnp.float32)
    wm_bd = (Wm[:, :, None] * eye[:, None, :]).reshape(N_VARS * GAMMA_DIM,
                                                       N_VARS)
    wv_bd = (Wv[:, :, None] * eye[:, None, :]).reshape(N_VARS * GAMMA_DIM,
                                                       N_VARS)
    whead = jnp.concatenate([wm_bd, wv_bd], axis=1)
    bias = jnp.concatenate([bm, bv]).reshape(1, 2 * N_VARS)

    G = N_VARS * GAMMA_DIM
    w2, bhead = pl.pallas_call(
        _fold_body,
        in_specs=[pl.BlockSpec((H_DIM, G), lambda: (0, 0)),
                  pl.BlockSpec((G, 2 * N_VARS), lambda: (0, 0)),
                  pl.BlockSpec((1, G), lambda: (0, 0)),
                  pl.BlockSpec((1, 2 * N_VARS), lambda: (0, 0))],
        out_specs=[pl.BlockSpec((H_DIM, 2 * N_VARS), lambda: (0, 0)),
                   pl.BlockSpec((1, 2 * N_VARS), lambda: (0, 0))],
        out_shape=[jax.ShapeDtypeStruct((H_DIM, 2 * N_VARS), jnp.float32),
                   jax.ShapeDtypeStruct((1, 2 * N_VARS), jnp.float32)],
    )(Wg, whead, bg.reshape(1, G), bias)

    grid = (B // BM,)
    row = lambda i: (i, 0)
    const = lambda i: (0, 0)
    out_specs = [pl.BlockSpec((BM, N_VARS), row) for _ in range(4)]
    out_shapes = [jax.ShapeDtypeStruct((B, N_VARS), jnp.float32)
                  for _ in range(4)]

    lp, lpm, est_mean, est_var = pl.pallas_call(
        _body,
        grid=grid,
        in_specs=[
            pl.BlockSpec((BM, Z_DIM), row),
            pl.BlockSpec((BM, N_VARS), row),
            pl.BlockSpec((BM, N_VARS), row),
            pl.BlockSpec((Z_DIM, H_DIM), const),
            pl.BlockSpec((1, H_DIM), const),
            pl.BlockSpec((H_DIM, 2 * N_VARS), const),
            pl.BlockSpec((1, 2 * N_VARS), const),
            pl.BlockSpec((1, N_VARS), const),
            pl.BlockSpec((1, N_VARS), const),
        ],
        out_specs=out_specs,
        out_shape=out_shapes,
    )(z, batch_x, miss_list,
      Wh, bh.reshape(1, H_DIM), w2, bhead,
      norm_params[:, 0].reshape(1, N_VARS), norm_params[:, 1].reshape(1, N_VARS))

    samples_x = est_mean
    params_x = jnp.stack([est_mean, est_var], axis=-1)
    return (lp, lpm, samples_x, params_x)
